# baseline (speedup 1.0000x reference)
import jax
import jax.numpy as jnp
from jax.experimental import pallas as pl
from jax.experimental.pallas import tpu as pltpu

_B, _N, _C = 16, 150, 64
_E = _N * _N
_I = 150
_T = _N // _I
_RPN = _N // 2
_ROWS = _I * _RPN
_C2 = 2 * _C


def _gnn_kernel(x_ref, x2_ref, e_ref, e2_ref, W_edge_ref, b_edge_ref, W_pre_ref,
                b_pre_ref, W_post_ref, b_post_ref, W_lin_ref, b_lin_ref,
                gn_w_ref, gn_b_ref, gn_ms_ref, out_ref,
                bd2_scr, bdc_scr, c2_scr, s1_scr, s2_scr, mn_scr, mx_scr):
    b = pl.program_id(0)
    t = pl.program_id(1)

    @pl.when(jnp.logical_and(b == 0, t == 0))
    def _init_weights():
        W2 = W_pre_ref[_C:2 * _C, :]
        W3 = W_pre_ref[2 * _C:3 * _C, :]
        Wc = jnp.dot(W_edge_ref[:], W3, preferred_element_type=jnp.float32)
        z = jnp.zeros((_C, _C), jnp.float32)
        bd2_scr[:] = jnp.concatenate(
            [jnp.concatenate([W2, z], axis=1),
             jnp.concatenate([z, W2], axis=1)], axis=0)
        bdc_scr[:] = jnp.concatenate(
            [jnp.concatenate([Wc, z], axis=1),
             jnp.concatenate([z, Wc], axis=1)], axis=0).astype(jnp.bfloat16)

    @pl.when(t == 0)
    def _per_batch():
        c2_scr[:] = jnp.dot(x2_ref[0], bd2_scr[:],
                            preferred_element_type=jnp.float32)

    c2 = c2_scr[:]
    for i in range(_I):
        src = e_ref if i < _I // 2 else e2_ref
        g = jnp.dot(src[0, 0, i % (_I // 2)].astype(jnp.bfloat16),
                    bdc_scr[:], preferred_element_type=jnp.float32)
        u = g + c2
        s1_scr[t, pl.ds(i, 1), :] = jnp.sum(u, axis=0, keepdims=True)
        s2_scr[t, pl.ds(i, 1), :] = jnp.sum(u * u, axis=0, keepdims=True)
        mn_scr[t, pl.ds(i, 1), :] = jnp.min(u, axis=0, keepdims=True)
        mx_scr[t, pl.ds(i, 1), :] = jnp.max(u, axis=0, keepdims=True)

    @pl.when(t == _T - 1)
    def _epilogue():
        inv_n = 1.0 / _N
        S1p = s1_scr[:].reshape(_N, _C2)
        S2p = s2_scr[:].reshape(_N, _C2)
        MNp = mn_scr[:].reshape(_N, _C2)
        MXp = mx_scr[:].reshape(_N, _C2)
        S1 = (S1p[:, :_C] + S1p[:, _C:]) * inv_n
        S2 = (S2p[:, :_C] + S2p[:, _C:]) * inv_n
        W1 = W_pre_ref[0:_C, :]
        W3 = W_pre_ref[2 * _C:3 * _C, :]
        X = x_ref[0]
        a = jnp.dot(X, W1, preferred_element_type=jnp.float32)
        const = (jnp.dot(b_edge_ref[:], W3,
                         preferred_element_type=jnp.float32)
                 + b_pre_ref[:])
        K = a + const
        mean = S1 + K
        mnv = jnp.minimum(MNp[:, :_C], MNp[:, _C:]) + K
        mxv = jnp.maximum(MXp[:, :_C], MXp[:, _C:]) + K
        var = S2 - S1 * S1
        std = jnp.sqrt(jnp.maximum(var, 0.0) + 1e-5)

        o = (jnp.dot(X, W_post_ref[0:_C, :],
                     preferred_element_type=jnp.float32)
             + jnp.dot(mean, W_post_ref[_C:2 * _C, :],
                       preferred_element_type=jnp.float32)
             + jnp.dot(mnv, W_post_ref[2 * _C:3 * _C, :],
                       preferred_element_type=jnp.float32)
             + jnp.dot(mxv, W_post_ref[3 * _C:4 * _C, :],
                       preferred_element_type=jnp.float32)
             + jnp.dot(std, W_post_ref[4 * _C:5 * _C, :],
                       preferred_element_type=jnp.float32)
             + b_post_ref[:])
        o = jnp.dot(o, W_lin_ref[:], preferred_element_type=jnp.float32) \
            + b_lin_ref[:]

        gmean = jnp.mean(o, axis=0, keepdims=True)
        cen = o - gmean * gn_ms_ref[:]
        gvar = jnp.mean(cen * cen, axis=0, keepdims=True)
        hn = gn_w_ref[:] * (cen * jax.lax.rsqrt(gvar + 1e-5)) + gn_b_ref[:]
        out_ref[0] = jnp.maximum(hn, 0.0) + X


def kernel(x, edge, W_edge, b_edge, W_pre, b_pre, W_post, b_post,
           W_lin, b_lin, gn_weight, gn_bias, gn_mean_scale):
    x2 = x.reshape(_B, _RPN, _C2)
    edge4 = edge.reshape(_B, 2, _I // 2, _RPN, _C2)
    row = lambda v: v.reshape(1, _C)

    return pl.pallas_call(
        _gnn_kernel,
        grid=(_B, _T),
        in_specs=[
            pl.BlockSpec((1, _N, _C), lambda b, t: (b, 0, 0)),
            pl.BlockSpec((1, _RPN, _C2), lambda b, t: (b, 0, 0)),
            pl.BlockSpec((1, 1, _I // 2, _RPN, _C2),
                         lambda b, t: (b, 0, 0, 0, 0)),
            pl.BlockSpec((1, 1, _I // 2, _RPN, _C2),
                         lambda b, t: (b, 1, 0, 0, 0)),
            pl.BlockSpec((_C, _C), lambda b, t: (0, 0)),
            pl.BlockSpec((1, _C), lambda b, t: (0, 0)),
            pl.BlockSpec((3 * _C, _C), lambda b, t: (0, 0)),
            pl.BlockSpec((1, _C), lambda b, t: (0, 0)),
            pl.BlockSpec((5 * _C, _C), lambda b, t: (0, 0)),
            pl.BlockSpec((1, _C), lambda b, t: (0, 0)),
            pl.BlockSpec((_C, _C), lambda b, t: (0, 0)),
            pl.BlockSpec((1, _C), lambda b, t: (0, 0)),
            pl.BlockSpec((1, _C), lambda b, t: (0, 0)),
            pl.BlockSpec((1, _C), lambda b, t: (0, 0)),
            pl.BlockSpec((1, _C), lambda b, t: (0, 0)),
        ],
        out_specs=pl.BlockSpec((1, _N, _C), lambda b, t: (b, 0, 0)),
        out_shape=jax.ShapeDtypeStruct((_B, _N, _C), jnp.float32),
        scratch_shapes=[
            pltpu.VMEM((_C2, _C2), jnp.float32),
            pltpu.VMEM((_C2, _C2), jnp.bfloat16),
            pltpu.VMEM((_RPN, _C2), jnp.float32),
            pltpu.VMEM((_T, _I, _C2), jnp.float32),
            pltpu.VMEM((_T, _I, _C2), jnp.float32),
            pltpu.VMEM((_T, _I, _C2), jnp.float32),
            pltpu.VMEM((_T, _I, _C2), jnp.float32),
        ],
    )(x, x2, edge4, edge4, W_edge, row(b_edge), W_pre, row(b_pre), W_post,
      row(b_post), W_lin, row(b_lin), row(gn_weight), row(gn_bias),
      row(gn_mean_scale))


# speedup vs baseline: 1.0124x; 1.0124x over previous
import jax
import jax.numpy as jnp
from jax.experimental import pallas as pl
from jax.experimental.pallas import tpu as pltpu

_B, _N, _C = 16, 150, 64
_RPN = _N // 2
_H = _N // 2
_C2 = 2 * _C


def _gnn_kernel(x_ref, x2_ref, e_ref, e2_ref, W_edge_ref, b_edge_ref,
                W_pre_ref, b_pre_ref, W_post_ref, b_post_ref, W_lin_ref,
                b_lin_ref, gn_w_ref, gn_b_ref, gn_ms_ref, out_ref,
                bd2_scr, bdc_scr, s1_scr, s2_scr, mn_scr, mx_scr):
    b = pl.program_id(0)

    @pl.when(b == 0)
    def _init_weights():
        W2 = W_pre_ref[_C:2 * _C, :]
        W3 = W_pre_ref[2 * _C:3 * _C, :]
        Wc = jnp.dot(W_edge_ref[:], W3, preferred_element_type=jnp.float32)
        z = jnp.zeros((_C, _C), jnp.float32)
        bd2_scr[:] = jnp.concatenate(
            [jnp.concatenate([W2, z], axis=1),
             jnp.concatenate([z, W2], axis=1)], axis=0)
        bdc_scr[:] = jnp.concatenate(
            [jnp.concatenate([Wc, z], axis=1),
             jnp.concatenate([z, Wc], axis=1)], axis=0)

    c2 = jnp.dot(x2_ref[0], bd2_scr[:], preferred_element_type=jnp.float32)

    for i in range(_N):
        src = e_ref if i < _H else e2_ref
        g = jnp.dot(src[0, 0, i % _H], bdc_scr[:],
                    preferred_element_type=jnp.float32)
        u = g + c2
        s1_scr[pl.ds(i, 1), :] = jnp.sum(u, axis=0, keepdims=True)
        s2_scr[pl.ds(i, 1), :] = jnp.sum(u * u, axis=0, keepdims=True)
        mn_scr[pl.ds(i, 1), :] = jnp.min(u, axis=0, keepdims=True)
        mx_scr[pl.ds(i, 1), :] = jnp.max(u, axis=0, keepdims=True)

    inv_n = 1.0 / _N
    S1p = s1_scr[:]
    S2p = s2_scr[:]
    MNp = mn_scr[:]
    MXp = mx_scr[:]
    S1 = (S1p[:, :_C] + S1p[:, _C:]) * inv_n
    S2 = (S2p[:, :_C] + S2p[:, _C:]) * inv_n
    W1 = W_pre_ref[0:_C, :]
    W3 = W_pre_ref[2 * _C:3 * _C, :]
    X = x_ref[0]
    a = jnp.dot(X, W1, preferred_element_type=jnp.float32)
    const = (jnp.dot(b_edge_ref[:], W3, preferred_element_type=jnp.float32)
             + b_pre_ref[:])
    K = a + const
    mean = S1 + K
    mnv = jnp.minimum(MNp[:, :_C], MNp[:, _C:]) + K
    mxv = jnp.maximum(MXp[:, :_C], MXp[:, _C:]) + K
    var = S2 - S1 * S1
    std = jnp.sqrt(jnp.maximum(var, 0.0) + 1e-5)

    o = (jnp.dot(X, W_post_ref[0:_C, :],
                 preferred_element_type=jnp.float32)
         + jnp.dot(mean, W_post_ref[_C:2 * _C, :],
                   preferred_element_type=jnp.float32)
         + jnp.dot(mnv, W_post_ref[2 * _C:3 * _C, :],
                   preferred_element_type=jnp.float32)
         + jnp.dot(mxv, W_post_ref[3 * _C:4 * _C, :],
                   preferred_element_type=jnp.float32)
         + jnp.dot(std, W_post_ref[4 * _C:5 * _C, :],
                   preferred_element_type=jnp.float32)
         + b_post_ref[:])
    o = jnp.dot(o, W_lin_ref[:], preferred_element_type=jnp.float32) \
        + b_lin_ref[:]

    gmean = jnp.mean(o, axis=0, keepdims=True)
    cen = o - gmean * gn_ms_ref[:]
    gvar = jnp.mean(cen * cen, axis=0, keepdims=True)
    hn = gn_w_ref[:] * (cen * jax.lax.rsqrt(gvar + 1e-5)) + gn_b_ref[:]
    out_ref[0] = jnp.maximum(hn, 0.0) + X


def kernel(x, edge, W_edge, b_edge, W_pre, b_pre, W_post, b_post,
           W_lin, b_lin, gn_weight, gn_bias, gn_mean_scale):
    x2 = x.reshape(_B, _RPN, _C2)
    edge5 = edge.reshape(_B, 2, _H, _RPN, _C2)
    row = lambda v: v.reshape(1, _C)

    return pl.pallas_call(
        _gnn_kernel,
        grid=(_B,),
        in_specs=[
            pl.BlockSpec((1, _N, _C), lambda b: (b, 0, 0)),
            pl.BlockSpec((1, _RPN, _C2), lambda b: (b, 0, 0)),
            pl.BlockSpec((1, 1, _H, _RPN, _C2),
                         lambda b: (b, 0, 0, 0, 0)),
            pl.BlockSpec((1, 1, _H, _RPN, _C2),
                         lambda b: (b, 1, 0, 0, 0)),
            pl.BlockSpec((_C, _C), lambda b: (0, 0)),
            pl.BlockSpec((1, _C), lambda b: (0, 0)),
            pl.BlockSpec((3 * _C, _C), lambda b: (0, 0)),
            pl.BlockSpec((1, _C), lambda b: (0, 0)),
            pl.BlockSpec((5 * _C, _C), lambda b: (0, 0)),
            pl.BlockSpec((1, _C), lambda b: (0, 0)),
            pl.BlockSpec((_C, _C), lambda b: (0, 0)),
            pl.BlockSpec((1, _C), lambda b: (0, 0)),
            pl.BlockSpec((1, _C), lambda b: (0, 0)),
            pl.BlockSpec((1, _C), lambda b: (0, 0)),
            pl.BlockSpec((1, _C), lambda b: (0, 0)),
        ],
        out_specs=pl.BlockSpec((1, _N, _C), lambda b: (b, 0, 0)),
        out_shape=jax.ShapeDtypeStruct((_B, _N, _C), jnp.float32),
        scratch_shapes=[
            pltpu.VMEM((_C2, _C2), jnp.float32),
            pltpu.VMEM((_C2, _C2), jnp.float32),
            pltpu.VMEM((_N, _C2), jnp.float32),
            pltpu.VMEM((_N, _C2), jnp.float32),
            pltpu.VMEM((_N, _C2), jnp.float32),
            pltpu.VMEM((_N, _C2), jnp.float32),
        ],
    )(x, x2, edge5, edge5, W_edge, row(b_edge), W_pre, row(b_pre), W_post,
      row(b_post), W_lin, row(b_lin), row(gn_weight), row(gn_bias),
      row(gn_mean_scale))


# speedup vs baseline: 1.0372x; 1.0245x over previous
import jax
import jax.numpy as jnp
from jax.experimental import pallas as pl
from jax.experimental.pallas import tpu as pltpu

_B, _N, _C = 16, 150, 64
_RPN = _N // 2
_H = _N // 2
_C2 = 2 * _C


def _gnn_kernel(x_ref, x2_ref, e_ref, e2_ref, W_edge_ref, b_edge_ref,
                W_pre_ref, b_pre_ref, W_post_ref, b_post_ref, W_lin_ref,
                b_lin_ref, gn_w_ref, gn_b_ref, gn_ms_ref, out_ref,
                bd2_scr, bdc_scr, s1_scr, s2_scr, mn_scr, mx_scr):
    b = pl.program_id(0)

    @pl.when(b == 0)
    def _init_weights():
        W2 = W_pre_ref[_C:2 * _C, :]
        W3 = W_pre_ref[2 * _C:3 * _C, :]
        Wc = jnp.dot(W_edge_ref[:], W3, preferred_element_type=jnp.float32)
        z = jnp.zeros((_C, _C), jnp.float32)
        bd2_scr[:] = jnp.concatenate(
            [jnp.concatenate([W2, z], axis=1),
             jnp.concatenate([z, W2], axis=1)], axis=0)
        bdc_scr[:] = jnp.concatenate(
            [jnp.concatenate([Wc, z], axis=1),
             jnp.concatenate([z, Wc], axis=1)], axis=0)

    c2 = jnp.dot(x2_ref[0], bd2_scr[:], preferred_element_type=jnp.float32)

    for i in range(_N):
        src = e_ref if i < _H else e2_ref
        r0 = (i % _H) * _RPN
        g = jnp.dot(src[0, 0, r0:r0 + _RPN, :], bdc_scr[:],
                    preferred_element_type=jnp.float32)
        u = g + c2
        s1_scr[pl.ds(i, 1), :] = jnp.sum(u, axis=0, keepdims=True)
        s2_scr[pl.ds(i, 1), :] = jnp.sum(u * u, axis=0, keepdims=True)
        mn_scr[pl.ds(i, 1), :] = jnp.min(u, axis=0, keepdims=True)
        mx_scr[pl.ds(i, 1), :] = jnp.max(u, axis=0, keepdims=True)

    inv_n = 1.0 / _N
    S1p = s1_scr[:]
    S2p = s2_scr[:]
    MNp = mn_scr[:]
    MXp = mx_scr[:]
    S1 = (S1p[:, :_C] + S1p[:, _C:]) * inv_n
    S2 = (S2p[:, :_C] + S2p[:, _C:]) * inv_n
    W1 = W_pre_ref[0:_C, :]
    W3 = W_pre_ref[2 * _C:3 * _C, :]
    X = x_ref[0]
    a = jnp.dot(X, W1, preferred_element_type=jnp.float32)
    const = (jnp.dot(b_edge_ref[:], W3, preferred_element_type=jnp.float32)
             + b_pre_ref[:])
    K = a + const
    mean = S1 + K
    mnv = jnp.minimum(MNp[:, :_C], MNp[:, _C:]) + K
    mxv = jnp.maximum(MXp[:, :_C], MXp[:, _C:]) + K
    var = S2 - S1 * S1
    std = jnp.sqrt(jnp.maximum(var, 0.0) + 1e-5)

    o = (jnp.dot(X, W_post_ref[0:_C, :],
                 preferred_element_type=jnp.float32)
         + jnp.dot(mean, W_post_ref[_C:2 * _C, :],
                   preferred_element_type=jnp.float32)
         + jnp.dot(mnv, W_post_ref[2 * _C:3 * _C, :],
                   preferred_element_type=jnp.float32)
         + jnp.dot(mxv, W_post_ref[3 * _C:4 * _C, :],
                   preferred_element_type=jnp.float32)
         + jnp.dot(std, W_post_ref[4 * _C:5 * _C, :],
                   preferred_element_type=jnp.float32)
         + b_post_ref[:])
    o = jnp.dot(o, W_lin_ref[:], preferred_element_type=jnp.float32) \
        + b_lin_ref[:]

    gmean = jnp.mean(o, axis=0, keepdims=True)
    cen = o - gmean * gn_ms_ref[:]
    gvar = jnp.mean(cen * cen, axis=0, keepdims=True)
    hn = gn_w_ref[:] * (cen * jax.lax.rsqrt(gvar + 1e-5)) + gn_b_ref[:]
    out_ref[0] = jnp.maximum(hn, 0.0) + X


def kernel(x, edge, W_edge, b_edge, W_pre, b_pre, W_post, b_post,
           W_lin, b_lin, gn_weight, gn_bias, gn_mean_scale):
    x2 = x.reshape(_B, _RPN, _C2)
    edge5 = edge.reshape(_B, 2, _H * _RPN, _C2)
    row = lambda v: v.reshape(1, _C)

    return pl.pallas_call(
        _gnn_kernel,
        grid=(_B,),
        in_specs=[
            pl.BlockSpec((1, _N, _C), lambda b: (b, 0, 0)),
            pl.BlockSpec((1, _RPN, _C2), lambda b: (b, 0, 0)),
            pl.BlockSpec((1, 1, _H * _RPN, _C2),
                         lambda b: (b, 0, 0, 0)),
            pl.BlockSpec((1, 1, _H * _RPN, _C2),
                         lambda b: (b, 1, 0, 0)),
            pl.BlockSpec((_C, _C), lambda b: (0, 0)),
            pl.BlockSpec((1, _C), lambda b: (0, 0)),
            pl.BlockSpec((3 * _C, _C), lambda b: (0, 0)),
            pl.BlockSpec((1, _C), lambda b: (0, 0)),
            pl.BlockSpec((5 * _C, _C), lambda b: (0, 0)),
            pl.BlockSpec((1, _C), lambda b: (0, 0)),
            pl.BlockSpec((_C, _C), lambda b: (0, 0)),
            pl.BlockSpec((1, _C), lambda b: (0, 0)),
            pl.BlockSpec((1, _C), lambda b: (0, 0)),
            pl.BlockSpec((1, _C), lambda b: (0, 0)),
            pl.BlockSpec((1, _C), lambda b: (0, 0)),
        ],
        out_specs=pl.BlockSpec((1, _N, _C), lambda b: (b, 0, 0)),
        out_shape=jax.ShapeDtypeStruct((_B, _N, _C), jnp.float32),
        scratch_shapes=[
            pltpu.VMEM((_C2, _C2), jnp.float32),
            pltpu.VMEM((_C2, _C2), jnp.float32),
            pltpu.VMEM((_N, _C2), jnp.float32),
            pltpu.VMEM((_N, _C2), jnp.float32),
            pltpu.VMEM((_N, _C2), jnp.float32),
            pltpu.VMEM((_N, _C2), jnp.float32),
        ],
    )(x, x2, edge5, edge5, W_edge, row(b_edge), W_pre, row(b_pre), W_post,
      row(b_post), W_lin, row(b_lin), row(gn_weight), row(gn_bias),
      row(gn_mean_scale))


# speedup vs baseline: 1.0395x; 1.0022x over previous
import jax
import jax.numpy as jnp
from jax.experimental import pallas as pl
from jax.experimental.pallas import tpu as pltpu

_B, _N, _C = 16, 150, 64
_RPN = _N // 2
_H = _N // 2
_C2 = 2 * _C


def _gnn_kernel(x_ref, x2_ref, e_ref, W_edge_ref, b_edge_ref,
                W_pre_ref, b_pre_ref, W_post_ref, b_post_ref, W_lin_ref,
                b_lin_ref, gn_w_ref, gn_b_ref, gn_ms_ref, out_ref,
                bd2_scr, bdc_scr, s1_scr, s2_scr, mn_scr, mx_scr):
    b = pl.program_id(0)

    @pl.when(b == 0)
    def _init_weights():
        W2 = W_pre_ref[_C:2 * _C, :]
        W3 = W_pre_ref[2 * _C:3 * _C, :]
        Wc = jnp.dot(W_edge_ref[:], W3, preferred_element_type=jnp.float32)
        z = jnp.zeros((_C, _C), jnp.float32)
        bd2_scr[:] = jnp.concatenate(
            [jnp.concatenate([W2, z], axis=1),
             jnp.concatenate([z, W2], axis=1)], axis=0)
        bdc_scr[:] = jnp.concatenate(
            [jnp.concatenate([Wc, z], axis=1),
             jnp.concatenate([z, Wc], axis=1)], axis=0)

    c2 = jnp.dot(x2_ref[0], bd2_scr[:], preferred_element_type=jnp.float32)

    for i in range(_N):
        r0 = i * _RPN
        g = jnp.dot(e_ref[0, r0:r0 + _RPN, :], bdc_scr[:],
                    preferred_element_type=jnp.float32)
        u = g + c2
        s1_scr[pl.ds(i, 1), :] = jnp.sum(u, axis=0, keepdims=True)
        s2_scr[pl.ds(i, 1), :] = jnp.sum(u * u, axis=0, keepdims=True)
        mn_scr[pl.ds(i, 1), :] = jnp.min(u, axis=0, keepdims=True)
        mx_scr[pl.ds(i, 1), :] = jnp.max(u, axis=0, keepdims=True)

    inv_n = 1.0 / _N
    S1p = s1_scr[:]
    S2p = s2_scr[:]
    MNp = mn_scr[:]
    MXp = mx_scr[:]
    S1 = (S1p[:, :_C] + S1p[:, _C:]) * inv_n
    S2 = (S2p[:, :_C] + S2p[:, _C:]) * inv_n
    W1 = W_pre_ref[0:_C, :]
    W3 = W_pre_ref[2 * _C:3 * _C, :]
    X = x_ref[0]
    a = jnp.dot(X, W1, preferred_element_type=jnp.float32)
    const = (jnp.dot(b_edge_ref[:], W3, preferred_element_type=jnp.float32)
             + b_pre_ref[:])
    K = a + const
    mean = S1 + K
    mnv = jnp.minimum(MNp[:, :_C], MNp[:, _C:]) + K
    mxv = jnp.maximum(MXp[:, :_C], MXp[:, _C:]) + K
    var = S2 - S1 * S1
    std = jnp.sqrt(jnp.maximum(var, 0.0) + 1e-5)

    o = (jnp.dot(X, W_post_ref[0:_C, :],
                 preferred_element_type=jnp.float32)
         + jnp.dot(mean, W_post_ref[_C:2 * _C, :],
                   preferred_element_type=jnp.float32)
         + jnp.dot(mnv, W_post_ref[2 * _C:3 * _C, :],
                   preferred_element_type=jnp.float32)
         + jnp.dot(mxv, W_post_ref[3 * _C:4 * _C, :],
                   preferred_element_type=jnp.float32)
         + jnp.dot(std, W_post_ref[4 * _C:5 * _C, :],
                   preferred_element_type=jnp.float32)
         + b_post_ref[:])
    o = jnp.dot(o, W_lin_ref[:], preferred_element_type=jnp.float32) \
        + b_lin_ref[:]

    gmean = jnp.mean(o, axis=0, keepdims=True)
    cen = o - gmean * gn_ms_ref[:]
    gvar = jnp.mean(cen * cen, axis=0, keepdims=True)
    hn = gn_w_ref[:] * (cen * jax.lax.rsqrt(gvar + 1e-5)) + gn_b_ref[:]
    out_ref[0] = jnp.maximum(hn, 0.0) + X


def kernel(x, edge, W_edge, b_edge, W_pre, b_pre, W_post, b_post,
           W_lin, b_lin, gn_weight, gn_bias, gn_mean_scale):
    x2 = x.reshape(_B, _RPN, _C2)
    edge5 = edge.reshape(_B, _N * _RPN, _C2)
    row = lambda v: v.reshape(1, _C)

    return pl.pallas_call(
        _gnn_kernel,
        grid=(_B,),
        in_specs=[
            pl.BlockSpec((1, _N, _C), lambda b: (b, 0, 0)),
            pl.BlockSpec((1, _RPN, _C2), lambda b: (b, 0, 0)),
            pl.BlockSpec((1, _N * _RPN, _C2),
                         lambda b: (b, 0, 0)),
            pl.BlockSpec((_C, _C), lambda b: (0, 0)),
            pl.BlockSpec((1, _C), lambda b: (0, 0)),
            pl.BlockSpec((3 * _C, _C), lambda b: (0, 0)),
            pl.BlockSpec((1, _C), lambda b: (0, 0)),
            pl.BlockSpec((5 * _C, _C), lambda b: (0, 0)),
            pl.BlockSpec((1, _C), lambda b: (0, 0)),
            pl.BlockSpec((_C, _C), lambda b: (0, 0)),
            pl.BlockSpec((1, _C), lambda b: (0, 0)),
            pl.BlockSpec((1, _C), lambda b: (0, 0)),
            pl.BlockSpec((1, _C), lambda b: (0, 0)),
            pl.BlockSpec((1, _C), lambda b: (0, 0)),
        ],
        out_specs=pl.BlockSpec((1, _N, _C), lambda b: (b, 0, 0)),
        out_shape=jax.ShapeDtypeStruct((_B, _N, _C), jnp.float32),
        scratch_shapes=[
            pltpu.VMEM((_C2, _C2), jnp.float32),
            pltpu.VMEM((_C2, _C2), jnp.float32),
            pltpu.VMEM((_N, _C2), jnp.float32),
            pltpu.VMEM((_N, _C2), jnp.float32),
            pltpu.VMEM((_N, _C2), jnp.float32),
            pltpu.VMEM((_N, _C2), jnp.float32),
        ],
    )(x, x2, edge5, W_edge, row(b_edge), W_pre, row(b_pre), W_post,
      row(b_post), W_lin, row(b_lin), row(gn_weight), row(gn_bias),
      row(gn_mean_scale))


# speedup vs baseline: 1.0426x; 1.0030x over previous
import jax
import jax.numpy as jnp
from jax.experimental import pallas as pl
from jax.experimental.pallas import tpu as pltpu

_B, _N, _C = 16, 150, 64
_RPN = _N // 2
_H = _N // 2
_C2 = 2 * _C


def _gnn_kernel(x_ref, x2_ref, e_ref, W_edge_ref, b_edge_ref,
                W_pre_ref, b_pre_ref, W_post_ref, b_post_ref, W_lin_ref,
                b_lin_ref, gn_w_ref, gn_b_ref, gn_ms_ref, out_ref,
                bd2_scr, bdc_scr, s1_scr, s2_scr, mn_scr, mx_scr):
    b = pl.program_id(0)

    @pl.when(b == 0)
    def _init_weights():
        W2 = W_pre_ref[_C:2 * _C, :]
        W3 = W_pre_ref[2 * _C:3 * _C, :]
        Wc = jnp.dot(W_edge_ref[:], W3, preferred_element_type=jnp.float32)
        z = jnp.zeros((_C, _C), jnp.float32)
        bd2_scr[:] = jnp.concatenate(
            [jnp.concatenate([W2, z], axis=1),
             jnp.concatenate([z, W2], axis=1)], axis=0)
        bdc_scr[:] = jnp.concatenate(
            [jnp.concatenate([Wc, z], axis=1),
             jnp.concatenate([z, Wc], axis=1)], axis=0).astype(jnp.bfloat16)

    c2 = jnp.dot(x2_ref[0], bd2_scr[:], preferred_element_type=jnp.float32)

    for i in range(_N):
        r0 = i * _RPN
        g = jnp.dot(e_ref[0, r0:r0 + _RPN, :].astype(jnp.bfloat16),
                    bdc_scr[:], preferred_element_type=jnp.float32)
        u = g + c2
        s1_scr[pl.ds(i, 1), :] = jnp.sum(u, axis=0, keepdims=True)
        s2_scr[pl.ds(i, 1), :] = jnp.sum(u * u, axis=0, keepdims=True)
        mn_scr[pl.ds(i, 1), :] = jnp.min(u, axis=0, keepdims=True)
        mx_scr[pl.ds(i, 1), :] = jnp.max(u, axis=0, keepdims=True)

    inv_n = 1.0 / _N
    S1p = s1_scr[:]
    S2p = s2_scr[:]
    MNp = mn_scr[:]
    MXp = mx_scr[:]
    S1 = (S1p[:, :_C] + S1p[:, _C:]) * inv_n
    S2 = (S2p[:, :_C] + S2p[:, _C:]) * inv_n
    W1 = W_pre_ref[0:_C, :]
    W3 = W_pre_ref[2 * _C:3 * _C, :]
    X = x_ref[0]
    a = jnp.dot(X, W1, preferred_element_type=jnp.float32)
    const = (jnp.dot(b_edge_ref[:], W3, preferred_element_type=jnp.float32)
             + b_pre_ref[:])
    K = a + const
    mean = S1 + K
    mnv = jnp.minimum(MNp[:, :_C], MNp[:, _C:]) + K
    mxv = jnp.maximum(MXp[:, :_C], MXp[:, _C:]) + K
    var = S2 - S1 * S1
    std = jnp.sqrt(jnp.maximum(var, 0.0) + 1e-5)

    o = (jnp.dot(X, W_post_ref[0:_C, :],
                 preferred_element_type=jnp.float32)
         + jnp.dot(mean, W_post_ref[_C:2 * _C, :],
                   preferred_element_type=jnp.float32)
         + jnp.dot(mnv, W_post_ref[2 * _C:3 * _C, :],
                   preferred_element_type=jnp.float32)
         + jnp.dot(mxv, W_post_ref[3 * _C:4 * _C, :],
                   preferred_element_type=jnp.float32)
         + jnp.dot(std, W_post_ref[4 * _C:5 * _C, :],
                   preferred_element_type=jnp.float32)
         + b_post_ref[:])
    o = jnp.dot(o, W_lin_ref[:], preferred_element_type=jnp.float32) \
        + b_lin_ref[:]

    gmean = jnp.mean(o, axis=0, keepdims=True)
    cen = o - gmean * gn_ms_ref[:]
    gvar = jnp.mean(cen * cen, axis=0, keepdims=True)
    hn = gn_w_ref[:] * (cen * jax.lax.rsqrt(gvar + 1e-5)) + gn_b_ref[:]
    out_ref[0] = jnp.maximum(hn, 0.0) + X


def kernel(x, edge, W_edge, b_edge, W_pre, b_pre, W_post, b_post,
           W_lin, b_lin, gn_weight, gn_bias, gn_mean_scale):
    x2 = x.reshape(_B, _RPN, _C2)
    edge5 = edge.reshape(_B, _N * _RPN, _C2)
    row = lambda v: v.reshape(1, _C)

    return pl.pallas_call(
        _gnn_kernel,
        grid=(_B,),
        in_specs=[
            pl.BlockSpec((1, _N, _C), lambda b: (b, 0, 0)),
            pl.BlockSpec((1, _RPN, _C2), lambda b: (b, 0, 0)),
            pl.BlockSpec((1, _N * _RPN, _C2),
                         lambda b: (b, 0, 0)),
            pl.BlockSpec((_C, _C), lambda b: (0, 0)),
            pl.BlockSpec((1, _C), lambda b: (0, 0)),
            pl.BlockSpec((3 * _C, _C), lambda b: (0, 0)),
            pl.BlockSpec((1, _C), lambda b: (0, 0)),
            pl.BlockSpec((5 * _C, _C), lambda b: (0, 0)),
            pl.BlockSpec((1, _C), lambda b: (0, 0)),
            pl.BlockSpec((_C, _C), lambda b: (0, 0)),
            pl.BlockSpec((1, _C), lambda b: (0, 0)),
            pl.BlockSpec((1, _C), lambda b: (0, 0)),
            pl.BlockSpec((1, _C), lambda b: (0, 0)),
            pl.BlockSpec((1, _C), lambda b: (0, 0)),
        ],
        out_specs=pl.BlockSpec((1, _N, _C), lambda b: (b, 0, 0)),
        out_shape=jax.ShapeDtypeStruct((_B, _N, _C), jnp.float32),
        scratch_shapes=[
            pltpu.VMEM((_C2, _C2), jnp.float32),
            pltpu.VMEM((_C2, _C2), jnp.bfloat16),
            pltpu.VMEM((_N, _C2), jnp.float32),
            pltpu.VMEM((_N, _C2), jnp.float32),
            pltpu.VMEM((_N, _C2), jnp.float32),
            pltpu.VMEM((_N, _C2), jnp.float32),
        ],
    )(x, x2, edge5, W_edge, row(b_edge), W_pre, row(b_pre), W_post,
      row(b_post), W_lin, row(b_lin), row(gn_weight), row(gn_bias),
      row(gn_mean_scale))


# speedup vs baseline: 1.0829x; 1.0387x over previous
import jax
import jax.numpy as jnp
from jax.experimental import pallas as pl
from jax.experimental.pallas import tpu as pltpu

_B, _N, _C = 16, 150, 64
_RPN = _N // 2
_G = 2
_C2 = 2 * _C


def _gnn_kernel(x_ref, x2_ref, e_ref, W_edge_ref, b_edge_ref,
                W_pre_ref, b_pre_ref, W_post_ref, b_post_ref, W_lin_ref,
                b_lin_ref, gn_w_ref, gn_b_ref, gn_ms_ref, out_ref,
                bd2_scr, bdc_scr, s1_scr, s2_scr, mn_scr, mx_scr):
    b = pl.program_id(0)

    @pl.when(b == 0)
    def _init_weights():
        W2 = W_pre_ref[_C:2 * _C, :]
        W3 = W_pre_ref[2 * _C:3 * _C, :]
        Wc = jnp.dot(W_edge_ref[:], W3, preferred_element_type=jnp.float32)
        z = jnp.zeros((_C, _C), jnp.float32)
        bd2_scr[:] = jnp.concatenate(
            [jnp.concatenate([W2, z], axis=1),
             jnp.concatenate([z, W2], axis=1)], axis=0)
        bdc_scr[:] = jnp.concatenate(
            [jnp.concatenate([Wc, z], axis=1),
             jnp.concatenate([z, Wc], axis=1)], axis=0).astype(jnp.bfloat16)

    for bb in range(_G):
        c2 = jnp.dot(x2_ref[0, bb], bd2_scr[:],
                     preferred_element_type=jnp.float32)
        base = bb * _N * _RPN
        for i in range(_N):
            r0 = base + i * _RPN
            g = jnp.dot(e_ref[0, r0:r0 + _RPN, :].astype(jnp.bfloat16),
                        bdc_scr[:], preferred_element_type=jnp.float32)
            u = g + c2
            s1_scr[bb, pl.ds(i, 1), :] = jnp.sum(u, axis=0, keepdims=True)
            s2_scr[bb, pl.ds(i, 1), :] = jnp.sum(u * u, axis=0,
                                                 keepdims=True)
            mn_scr[bb, pl.ds(i, 1), :] = jnp.min(u, axis=0, keepdims=True)
            mx_scr[bb, pl.ds(i, 1), :] = jnp.max(u, axis=0, keepdims=True)

    inv_n = 1.0 / _N
    W1 = W_pre_ref[0:_C, :]
    W3 = W_pre_ref[2 * _C:3 * _C, :]
    const = (jnp.dot(b_edge_ref[:], W3, preferred_element_type=jnp.float32)
             + b_pre_ref[:])
    for bb in range(_G):
        S1p = s1_scr[bb]
        S2p = s2_scr[bb]
        MNp = mn_scr[bb]
        MXp = mx_scr[bb]
        S1 = (S1p[:, :_C] + S1p[:, _C:]) * inv_n
        S2 = (S2p[:, :_C] + S2p[:, _C:]) * inv_n
        X = x_ref[0, bb]
        a = jnp.dot(X, W1, preferred_element_type=jnp.float32)
        K = a + const
        mean = S1 + K
        mnv = jnp.minimum(MNp[:, :_C], MNp[:, _C:]) + K
        mxv = jnp.maximum(MXp[:, :_C], MXp[:, _C:]) + K
        var = S2 - S1 * S1
        std = jnp.sqrt(jnp.maximum(var, 0.0) + 1e-5)

        o = (jnp.dot(X, W_post_ref[0:_C, :],
                     preferred_element_type=jnp.float32)
             + jnp.dot(mean, W_post_ref[_C:2 * _C, :],
                       preferred_element_type=jnp.float32)
             + jnp.dot(mnv, W_post_ref[2 * _C:3 * _C, :],
                       preferred_element_type=jnp.float32)
             + jnp.dot(mxv, W_post_ref[3 * _C:4 * _C, :],
                       preferred_element_type=jnp.float32)
             + jnp.dot(std, W_post_ref[4 * _C:5 * _C, :],
                       preferred_element_type=jnp.float32)
             + b_post_ref[:])
        o = jnp.dot(o, W_lin_ref[:], preferred_element_type=jnp.float32) \
            + b_lin_ref[:]

        gmean = jnp.mean(o, axis=0, keepdims=True)
        cen = o - gmean * gn_ms_ref[:]
        gvar = jnp.mean(cen * cen, axis=0, keepdims=True)
        hn = gn_w_ref[:] * (cen * jax.lax.rsqrt(gvar + 1e-5)) + gn_b_ref[:]
        out_ref[0, bb] = jnp.maximum(hn, 0.0) + X


def kernel(x, edge, W_edge, b_edge, W_pre, b_pre, W_post, b_post,
           W_lin, b_lin, gn_weight, gn_bias, gn_mean_scale):
    nb = _B // _G
    x4 = x.reshape(nb, _G, _N, _C)
    x2 = x.reshape(nb, _G, _RPN, _C2)
    edge5 = edge.reshape(nb, _G * _N * _RPN, _C2)
    row = lambda v: v.reshape(1, _C)

    out = pl.pallas_call(
        _gnn_kernel,
        grid=(nb,),
        in_specs=[
            pl.BlockSpec((1, _G, _N, _C), lambda b: (b, 0, 0, 0)),
            pl.BlockSpec((1, _G, _RPN, _C2),
                         lambda b: (b, 0, 0, 0)),
            pl.BlockSpec((1, _G * _N * _RPN, _C2),
                         lambda b: (b, 0, 0)),
            pl.BlockSpec((_C, _C), lambda b: (0, 0)),
            pl.BlockSpec((1, _C), lambda b: (0, 0)),
            pl.BlockSpec((3 * _C, _C), lambda b: (0, 0)),
            pl.BlockSpec((1, _C), lambda b: (0, 0)),
            pl.BlockSpec((5 * _C, _C), lambda b: (0, 0)),
            pl.BlockSpec((1, _C), lambda b: (0, 0)),
            pl.BlockSpec((_C, _C), lambda b: (0, 0)),
            pl.BlockSpec((1, _C), lambda b: (0, 0)),
            pl.BlockSpec((1, _C), lambda b: (0, 0)),
            pl.BlockSpec((1, _C), lambda b: (0, 0)),
            pl.BlockSpec((1, _C), lambda b: (0, 0)),
        ],
        out_specs=pl.BlockSpec((1, _G, _N, _C), lambda b: (b, 0, 0, 0)),
        out_shape=jax.ShapeDtypeStruct((nb, _G, _N, _C), jnp.float32),
        scratch_shapes=[
            pltpu.VMEM((_C2, _C2), jnp.float32),
            pltpu.VMEM((_C2, _C2), jnp.bfloat16),
            pltpu.VMEM((_G, _N, _C2), jnp.float32),
            pltpu.VMEM((_G, _N, _C2), jnp.float32),
            pltpu.VMEM((_G, _N, _C2), jnp.float32),
            pltpu.VMEM((_G, _N, _C2), jnp.float32),
        ],
    )(x4, x2, edge5, W_edge, row(b_edge), W_pre, row(b_pre), W_post,
      row(b_post), W_lin, row(b_lin), row(gn_weight), row(gn_bias),
      row(gn_mean_scale))
    return out.reshape(_B, _N, _C)


# speedup vs baseline: 1.0840x; 1.0010x over previous
import jax
import jax.numpy as jnp
from jax.experimental import pallas as pl
from jax.experimental.pallas import tpu as pltpu

_B, _N, _C = 16, 150, 64
_RPN = _N // 2
_G = 4
_C2 = 2 * _C


def _gnn_kernel(x_ref, x2_ref, e_ref, W_edge_ref, b_edge_ref,
                W_pre_ref, b_pre_ref, W_post_ref, b_post_ref, W_lin_ref,
                b_lin_ref, gn_w_ref, gn_b_ref, gn_ms_ref, out_ref,
                bd2_scr, bdc_scr, s1_scr, s2_scr, mn_scr, mx_scr):
    b = pl.program_id(0)

    @pl.when(b == 0)
    def _init_weights():
        W2 = W_pre_ref[_C:2 * _C, :]
        W3 = W_pre_ref[2 * _C:3 * _C, :]
        Wc = jnp.dot(W_edge_ref[:], W3, preferred_element_type=jnp.float32)
        z = jnp.zeros((_C, _C), jnp.float32)
        bd2_scr[:] = jnp.concatenate(
            [jnp.concatenate([W2, z], axis=1),
             jnp.concatenate([z, W2], axis=1)], axis=0)
        bdc_scr[:] = jnp.concatenate(
            [jnp.concatenate([Wc, z], axis=1),
             jnp.concatenate([z, Wc], axis=1)], axis=0).astype(jnp.bfloat16)

    for bb in range(_G):
        c2 = jnp.dot(x2_ref[0, bb], bd2_scr[:],
                     preferred_element_type=jnp.float32)
        base = bb * _N * _RPN
        for i in range(_N):
            r0 = base + i * _RPN
            g = jnp.dot(e_ref[0, r0:r0 + _RPN, :].astype(jnp.bfloat16),
                        bdc_scr[:], preferred_element_type=jnp.float32)
            u = g + c2
            s1_scr[bb, pl.ds(i, 1), :] = jnp.sum(u, axis=0, keepdims=True)
            s2_scr[bb, pl.ds(i, 1), :] = jnp.sum(u * u, axis=0,
                                                 keepdims=True)
            mn_scr[bb, pl.ds(i, 1), :] = jnp.min(u, axis=0, keepdims=True)
            mx_scr[bb, pl.ds(i, 1), :] = jnp.max(u, axis=0, keepdims=True)

    inv_n = 1.0 / _N
    W1 = W_pre_ref[0:_C, :]
    W3 = W_pre_ref[2 * _C:3 * _C, :]
    const = (jnp.dot(b_edge_ref[:], W3, preferred_element_type=jnp.float32)
             + b_pre_ref[:])
    for bb in range(_G):
        S1p = s1_scr[bb]
        S2p = s2_scr[bb]
        MNp = mn_scr[bb]
        MXp = mx_scr[bb]
        S1 = (S1p[:, :_C] + S1p[:, _C:]) * inv_n
        S2 = (S2p[:, :_C] + S2p[:, _C:]) * inv_n
        X = x_ref[0, bb]
        a = jnp.dot(X, W1, preferred_element_type=jnp.float32)
        K = a + const
        mean = S1 + K
        mnv = jnp.minimum(MNp[:, :_C], MNp[:, _C:]) + K
        mxv = jnp.maximum(MXp[:, :_C], MXp[:, _C:]) + K
        var = S2 - S1 * S1
        std = jnp.sqrt(jnp.maximum(var, 0.0) + 1e-5)

        o = (jnp.dot(X, W_post_ref[0:_C, :],
                     preferred_element_type=jnp.float32)
             + jnp.dot(mean, W_post_ref[_C:2 * _C, :],
                       preferred_element_type=jnp.float32)
             + jnp.dot(mnv, W_post_ref[2 * _C:3 * _C, :],
                       preferred_element_type=jnp.float32)
             + jnp.dot(mxv, W_post_ref[3 * _C:4 * _C, :],
                       preferred_element_type=jnp.float32)
             + jnp.dot(std, W_post_ref[4 * _C:5 * _C, :],
                       preferred_element_type=jnp.float32)
             + b_post_ref[:])
        o = jnp.dot(o, W_lin_ref[:], preferred_element_type=jnp.float32) \
            + b_lin_ref[:]

        gmean = jnp.mean(o, axis=0, keepdims=True)
        cen = o - gmean * gn_ms_ref[:]
        gvar = jnp.mean(cen * cen, axis=0, keepdims=True)
        hn = gn_w_ref[:] * (cen * jax.lax.rsqrt(gvar + 1e-5)) + gn_b_ref[:]
        out_ref[0, bb] = jnp.maximum(hn, 0.0) + X


def kernel(x, edge, W_edge, b_edge, W_pre, b_pre, W_post, b_post,
           W_lin, b_lin, gn_weight, gn_bias, gn_mean_scale):
    nb = _B // _G
    x4 = x.reshape(nb, _G, _N, _C)
    x2 = x.reshape(nb, _G, _RPN, _C2)
    edge5 = edge.reshape(nb, _G * _N * _RPN, _C2)
    row = lambda v: v.reshape(1, _C)

    out = pl.pallas_call(
        _gnn_kernel,
        grid=(nb,),
        in_specs=[
            pl.BlockSpec((1, _G, _N, _C), lambda b: (b, 0, 0, 0)),
            pl.BlockSpec((1, _G, _RPN, _C2),
                         lambda b: (b, 0, 0, 0)),
            pl.BlockSpec((1, _G * _N * _RPN, _C2),
                         lambda b: (b, 0, 0)),
            pl.BlockSpec((_C, _C), lambda b: (0, 0)),
            pl.BlockSpec((1, _C), lambda b: (0, 0)),
            pl.BlockSpec((3 * _C, _C), lambda b: (0, 0)),
            pl.BlockSpec((1, _C), lambda b: (0, 0)),
            pl.BlockSpec((5 * _C, _C), lambda b: (0, 0)),
            pl.BlockSpec((1, _C), lambda b: (0, 0)),
            pl.BlockSpec((_C, _C), lambda b: (0, 0)),
            pl.BlockSpec((1, _C), lambda b: (0, 0)),
            pl.BlockSpec((1, _C), lambda b: (0, 0)),
            pl.BlockSpec((1, _C), lambda b: (0, 0)),
            pl.BlockSpec((1, _C), lambda b: (0, 0)),
        ],
        out_specs=pl.BlockSpec((1, _G, _N, _C), lambda b: (b, 0, 0, 0)),
        out_shape=jax.ShapeDtypeStruct((nb, _G, _N, _C), jnp.float32),
        scratch_shapes=[
            pltpu.VMEM((_C2, _C2), jnp.float32),
            pltpu.VMEM((_C2, _C2), jnp.bfloat16),
            pltpu.VMEM((_G, _N, _C2), jnp.float32),
            pltpu.VMEM((_G, _N, _C2), jnp.float32),
            pltpu.VMEM((_G, _N, _C2), jnp.float32),
            pltpu.VMEM((_G, _N, _C2), jnp.float32),
        ],
    )(x4, x2, edge5, W_edge, row(b_edge), W_pre, row(b_pre), W_post,
      row(b_post), W_lin, row(b_lin), row(gn_weight), row(gn_bias),
      row(gn_mean_scale))
    return out.reshape(_B, _N, _C)
